# Initial kernel scaffold; baseline (speedup 1.0000x reference)
import jax
import jax.numpy as jnp
from jax.experimental import pallas as pl


def kernel(visited_node_score, node_repr, rel_emb, rule_prior, query_src_ts_emb, query_rel_emb, Wq, Wk, W_lin, b_lin, gamma, edges):
    raise NotImplementedError("write your pallas kernel here")


# speedup vs baseline: 1.1613x; 1.1613x over previous
import functools

import jax
import jax.numpy as jnp
from jax import lax
from jax.experimental import pallas as pl
from jax.experimental.pallas import tpu as pltpu
from jax.experimental.pallas import tpu_sc as plsc

_F32 = jnp.float32
_BF16 = jnp.bfloat16


def _sc_gather(hrep, qq, vi_p, vj_p, q_p, e_pad, nw, ch):
    per_w = e_pad // nw
    nchunk = per_w // ch
    nc = 2
    mesh = plsc.VectorSubcoreMesh(core_axis_name="c", subcore_axis_name="s")

    @functools.partial(
        pl.kernel,
        mesh=mesh,
        out_type=(
            jax.ShapeDtypeStruct((e_pad, 128), _F32),
            jax.ShapeDtypeStruct((e_pad, 128), _F32),
            jax.ShapeDtypeStruct((e_pad, 256), _F32),
        ),
        scratch_types=[
            pltpu.VMEM((ch,), jnp.int32),
            pltpu.VMEM((ch,), jnp.int32),
            pltpu.VMEM((ch,), jnp.int32),
            pltpu.VMEM((ch, 128), _F32),
            pltpu.VMEM((ch, 128), _F32),
            pltpu.VMEM((ch, 256), _F32),
            pltpu.SemaphoreType.DMA,
            pltpu.SemaphoreType.DMA,
            pltpu.SemaphoreType.DMA,
        ],
    )
    def gath(h_h, qq_h, vi_h, vj_h, q_h, gvi_h, gvj_h, gqq_h,
             ivi, ivj, iq, rvi, rvj, rqq, s1, s2, s3):
        wid = lax.axis_index("s") * nc + lax.axis_index("c")

        def step(c, carry):
            base = wid * per_w + c * ch
            pltpu.sync_copy(vi_h.at[pl.ds(base, ch)], ivi)
            pltpu.sync_copy(vj_h.at[pl.ds(base, ch)], ivj)
            pltpu.sync_copy(q_h.at[pl.ds(base, ch)], iq)
            cp1 = pltpu.async_copy(h_h.at[ivi], rvi, s1)
            cp2 = pltpu.async_copy(h_h.at[ivj], rvj, s2)
            cp3 = pltpu.async_copy(qq_h.at[iq], rqq, s3)
            cp1.wait()
            cp2.wait()
            cp3.wait()
            pltpu.sync_copy(rvi, gvi_h.at[pl.ds(base, ch)])
            pltpu.sync_copy(rvj, gvj_h.at[pl.ds(base, ch)])
            pltpu.sync_copy(rqq, gqq_h.at[pl.ds(base, ch)])
            return carry

        lax.fori_loop(0, nchunk, step, 0)

    return gath(hrep, qq, vi_p, vj_p, q_p)


def _pair_logits(gvi, gvj, gqq, rel_p, rp2, gamma11, Wq, Wk, e_pad, be):
    steps = e_pad // be
    dn = (((1,), (1,)), ((), ()))

    def body(gvi_ref, gvj_ref, gqq_ref, rel_ref, rp_ref, g_ref,
             wq_ref, wk_ref, out_ref):
        r = rel_ref[...].astype(_BF16)
        qqv = gqq_ref[...].astype(_BF16)
        left = jnp.concatenate([gvi_ref[...].astype(_BF16), r, qqv], axis=1)
        right = jnp.concatenate([gvj_ref[...].astype(_BF16), r, qqv], axis=1)
        lq = lax.dot_general(left, wq_ref[...].astype(_BF16), dn,
                             preferred_element_type=_F32)
        rk = lax.dot_general(right, wk_ref[...].astype(_BF16), dn,
                             preferred_element_type=_F32)
        base = jnp.sum(lq * rk, axis=1, keepdims=True)
        out_ref[...] = base + g_ref[0, 0] * rp_ref[...]

    return pl.pallas_call(
        body,
        grid=(steps,),
        in_specs=[
            pl.BlockSpec((be, 128), lambda i: (i, 0)),
            pl.BlockSpec((be, 128), lambda i: (i, 0)),
            pl.BlockSpec((be, 256), lambda i: (i, 0)),
            pl.BlockSpec((be, 128), lambda i: (i, 0)),
            pl.BlockSpec((be, 1), lambda i: (i, 0)),
            pl.BlockSpec((1, 1), lambda i: (0, 0)),
            pl.BlockSpec((512, 512), lambda i: (0, 0)),
            pl.BlockSpec((512, 512), lambda i: (0, 0)),
        ],
        out_specs=pl.BlockSpec((be, 1), lambda i: (i, 0)),
        out_shape=jax.ShapeDtypeStruct((e_pad, 1), _F32),
    )(gvi, gvj, gqq, rel_p, rp2, gamma11, Wq, Wk)


def _output_transform(agg, has2, h, wl, bl, n, bn):
    grid = n // bn

    def body(agg_ref, has_ref, h_ref, wl_ref, bl_ref, out_ref):
        hasv = has_ref[...] > 0.0
        u = jnp.where(hasv, agg_ref[...], h_ref[...])
        y = lax.dot_general(u.astype(_BF16), wl_ref[...].astype(_BF16),
                            (((1,), (1,)), ((), ())),
                            preferred_element_type=_F32) + bl_ref[...]
        out_ref[...] = jnp.where(y >= 0.0, y, 0.01 * y)

    return pl.pallas_call(
        body,
        grid=(grid,),
        in_specs=[
            pl.BlockSpec((bn, 128), lambda i: (i, 0)),
            pl.BlockSpec((bn, 1), lambda i: (i, 0)),
            pl.BlockSpec((bn, 128), lambda i: (i, 0)),
            pl.BlockSpec((128, 128), lambda i: (0, 0)),
            pl.BlockSpec((1, 128), lambda i: (0, 0)),
        ],
        out_specs=pl.BlockSpec((bn, 128), lambda i: (i, 0)),
        out_shape=jax.ShapeDtypeStruct((n, 128), _F32),
    )(agg, has2, h, wl, bl)


def kernel(visited_node_score, node_repr, rel_emb, rule_prior,
           query_src_ts_emb, query_rel_emb, Wq, Wk, W_lin, b_lin, gamma, edges):
    n, d = node_repr.shape
    e = rel_emb.shape[0]
    k = 10

    nw, ch = 32, 192
    e_pad = ((e + nw * ch - 1) // (nw * ch)) * nw * ch
    be = 1296
    assert e_pad % be == 0

    q_idx = edges[:, 0]
    vi = edges[:, 6]
    vj = edges[:, 7]
    pad = e_pad - e
    q_p = jnp.pad(q_idx, (0, pad))
    vi_p = jnp.pad(vi, (0, pad))
    vj_p = jnp.pad(vj, (0, pad))

    qq = jnp.concatenate([query_src_ts_emb, query_rel_emb], axis=1)
    gvi, gvj, gqq = _sc_gather(node_repr, qq, vi_p, vj_p, q_p, e_pad, nw, ch)

    rel_p = jnp.pad(rel_emb, ((0, pad), (0, 0)))
    rp2 = jnp.pad(rule_prior, (0, pad)).reshape(e_pad, 1)
    gamma11 = jnp.reshape(gamma.astype(_F32), (1, 1))
    lg2 = _pair_logits(gvi, gvj, gqq, rel_p, rp2, gamma11, Wq, Wk, e_pad, be)
    logits = lg2.reshape(e_pad)[:e]

    m = jax.ops.segment_max(logits, vi, num_segments=n)
    ex = jnp.exp(logits - m[vi])
    den = jax.ops.segment_sum(ex, vi, num_segments=n)
    soft = ex / den[vi]
    target = soft * visited_node_score[vi]

    order = jnp.lexsort((-target, q_idx))
    sorted_group = q_idx[order]
    counts = jnp.bincount(q_idx, length=n)
    starts = jnp.concatenate([jnp.zeros((1,), counts.dtype),
                              jnp.cumsum(counts)[:-1]])
    rank_sorted = jnp.arange(e) - starts[sorted_group]
    rank = jnp.zeros((e,), jnp.int32).at[order].set(rank_sorted.astype(jnp.int32))
    keep = (rank < k).astype(_F32)

    updated_node_score = jax.ops.segment_sum(target * keep, vj, num_segments=n)

    hvj_edge = gvj[:e, :]
    msg = (soft * keep)[:, None] * hvj_edge
    agg = jax.ops.segment_sum(msg, vi, num_segments=n)
    has = jax.ops.segment_sum(keep, vi, num_segments=n)

    has2 = has.reshape(n, 1)
    bl = b_lin.reshape(1, d)
    updated_repr = _output_transform(agg, has2, node_repr, W_lin, bl, n, 1000)

    return updated_node_score, updated_repr


# speedup vs baseline: 1.2876x; 1.1088x over previous
import functools

import jax
import jax.numpy as jnp
from jax import lax
from jax.experimental import pallas as pl
from jax.experimental.pallas import tpu as pltpu
from jax.experimental.pallas import tpu_sc as plsc

_F32 = jnp.float32
_BF16 = jnp.bfloat16


def _sc_gather(hrep, qq, vi_p, vj_p, q_p, e_pad, nw, ch):
    per_w = e_pad // nw
    nchunk = per_w // ch
    nc = 2
    mesh = plsc.VectorSubcoreMesh(core_axis_name="c", subcore_axis_name="s")

    @functools.partial(
        pl.kernel,
        mesh=mesh,
        out_type=(
            jax.ShapeDtypeStruct((e_pad, 128), _F32),
            jax.ShapeDtypeStruct((e_pad, 128), _F32),
            jax.ShapeDtypeStruct((e_pad, 256), _F32),
        ),
        scratch_types=[
            pltpu.VMEM((ch,), jnp.int32),
            pltpu.VMEM((ch,), jnp.int32),
            pltpu.VMEM((ch,), jnp.int32),
            pltpu.VMEM((ch, 128), _F32),
            pltpu.VMEM((ch, 128), _F32),
            pltpu.VMEM((ch, 256), _F32),
            pltpu.SemaphoreType.DMA,
            pltpu.SemaphoreType.DMA,
            pltpu.SemaphoreType.DMA,
        ],
    )
    def gath(h_h, qq_h, vi_h, vj_h, q_h, gvi_h, gvj_h, gqq_h,
             ivi, ivj, iq, rvi, rvj, rqq, s1, s2, s3):
        wid = lax.axis_index("s") * nc + lax.axis_index("c")

        def step(c, carry):
            base = wid * per_w + c * ch
            pltpu.sync_copy(vi_h.at[pl.ds(base, ch)], ivi)
            pltpu.sync_copy(vj_h.at[pl.ds(base, ch)], ivj)
            pltpu.sync_copy(q_h.at[pl.ds(base, ch)], iq)
            cp1 = pltpu.async_copy(h_h.at[ivi], rvi, s1)
            cp2 = pltpu.async_copy(h_h.at[ivj], rvj, s2)
            cp3 = pltpu.async_copy(qq_h.at[iq], rqq, s3)
            cp1.wait()
            cp2.wait()
            cp3.wait()
            pltpu.sync_copy(rvi, gvi_h.at[pl.ds(base, ch)])
            pltpu.sync_copy(rvj, gvj_h.at[pl.ds(base, ch)])
            pltpu.sync_copy(rqq, gqq_h.at[pl.ds(base, ch)])
            return carry

        lax.fori_loop(0, nchunk, step, 0)

    return gath(hrep, qq, vi_p, vj_p, q_p)


def _pair_logits(gvi, gvj, gqq, rel_p, rp2, gamma11, Wq, Wk, e_pad, be):
    steps = e_pad // be
    dn = (((1,), (1,)), ((), ()))

    def body(gvi_ref, gvj_ref, gqq_ref, rel_ref, rp_ref, g_ref,
             wq_ref, wk_ref, out_ref):
        r = rel_ref[...].astype(_BF16)
        qqv = gqq_ref[...].astype(_BF16)
        left = jnp.concatenate([gvi_ref[...].astype(_BF16), r, qqv], axis=1)
        right = jnp.concatenate([gvj_ref[...].astype(_BF16), r, qqv], axis=1)
        lq = lax.dot_general(left, wq_ref[...].astype(_BF16), dn,
                             preferred_element_type=_F32)
        rk = lax.dot_general(right, wk_ref[...].astype(_BF16), dn,
                             preferred_element_type=_F32)
        base = jnp.sum(lq * rk, axis=1, keepdims=True)
        out_ref[...] = base + g_ref[0, 0] * rp_ref[...]

    return pl.pallas_call(
        body,
        grid=(steps,),
        in_specs=[
            pl.BlockSpec((be, 128), lambda i: (i, 0)),
            pl.BlockSpec((be, 128), lambda i: (i, 0)),
            pl.BlockSpec((be, 256), lambda i: (i, 0)),
            pl.BlockSpec((be, 128), lambda i: (i, 0)),
            pl.BlockSpec((be, 1), lambda i: (i, 0)),
            pl.BlockSpec((1, 1), lambda i: (0, 0)),
            pl.BlockSpec((512, 512), lambda i: (0, 0)),
            pl.BlockSpec((512, 512), lambda i: (0, 0)),
        ],
        out_specs=pl.BlockSpec((be, 1), lambda i: (i, 0)),
        out_shape=jax.ShapeDtypeStruct((e_pad, 1), _F32),
    )(gvi, gvj, gqq, rel_p, rp2, gamma11, Wq, Wk)


def _output_transform(agg, has2, h, wl, bl, n, bn):
    grid = n // bn

    def body(agg_ref, has_ref, h_ref, wl_ref, bl_ref, out_ref):
        hasv = has_ref[...] > 0.0
        u = jnp.where(hasv, agg_ref[...], h_ref[...])
        y = lax.dot_general(u.astype(_BF16), wl_ref[...].astype(_BF16),
                            (((1,), (1,)), ((), ())),
                            preferred_element_type=_F32) + bl_ref[...]
        out_ref[...] = jnp.where(y >= 0.0, y, 0.01 * y)

    return pl.pallas_call(
        body,
        grid=(grid,),
        in_specs=[
            pl.BlockSpec((bn, 128), lambda i: (i, 0)),
            pl.BlockSpec((bn, 1), lambda i: (i, 0)),
            pl.BlockSpec((bn, 128), lambda i: (i, 0)),
            pl.BlockSpec((128, 128), lambda i: (0, 0)),
            pl.BlockSpec((1, 128), lambda i: (0, 0)),
        ],
        out_specs=pl.BlockSpec((bn, 128), lambda i: (i, 0)),
        out_shape=jax.ShapeDtypeStruct((n, 128), _F32),
    )(agg, has2, h, wl, bl)


def kernel(visited_node_score, node_repr, rel_emb, rule_prior,
           query_src_ts_emb, query_rel_emb, Wq, Wk, W_lin, b_lin, gamma, edges):
    n, d = node_repr.shape
    e = rel_emb.shape[0]
    k = 10

    nw, ch = 32, 192
    e_pad = ((e + nw * ch - 1) // (nw * ch)) * nw * ch
    be = 1296
    assert e_pad % be == 0

    q_idx = edges[:, 0]
    vi = edges[:, 6]
    vj = edges[:, 7]
    pad = e_pad - e
    q_p = jnp.pad(q_idx, (0, pad))
    vi_p = jnp.pad(vi, (0, pad))
    vj_p = jnp.pad(vj, (0, pad))

    qq = jnp.concatenate([query_src_ts_emb, query_rel_emb], axis=1)
    gvi, gvj, gqq = _sc_gather(node_repr, qq, vi_p, vj_p, q_p, e_pad, nw, ch)

    rel_p = jnp.pad(rel_emb, ((0, pad), (0, 0)))
    rp2 = jnp.pad(rule_prior, (0, pad)).reshape(e_pad, 1)
    gamma11 = jnp.reshape(gamma.astype(_F32), (1, 1))
    lg2 = _pair_logits(gvi, gvj, gqq, rel_p, rp2, gamma11, Wq, Wk, e_pad, be)
    logits = lg2.reshape(e_pad)[:e]

    m = jax.ops.segment_max(logits, vi, num_segments=n)
    ex = jnp.exp(logits - m[vi])
    den = jax.ops.segment_sum(ex, vi, num_segments=n)
    soft = ex / den[vi]
    target = soft * visited_node_score[vi]

    order = jnp.lexsort((-target, q_idx))
    sorted_target = target[order]
    counts = jnp.bincount(q_idx, length=n)
    starts = jnp.concatenate([jnp.zeros((1,), counts.dtype),
                              jnp.cumsum(counts)[:-1]])
    thr_pos = jnp.clip(starts + jnp.minimum(counts, k) - 1, 0, e - 1)
    thr = sorted_target[thr_pos]
    keep = (target >= thr[q_idx]).astype(_F32)

    updated_node_score = jax.ops.segment_sum(target * keep, vj, num_segments=n)

    hvj_edge = gvj[:e, :]
    msg = (soft * keep)[:, None] * hvj_edge
    agg = jax.ops.segment_sum(msg, vi, num_segments=n)
    has = jax.ops.segment_sum(keep, vi, num_segments=n)

    has2 = has.reshape(n, 1)
    bl = b_lin.reshape(1, d)
    updated_repr = _output_transform(agg, has2, node_repr, W_lin, bl, n, 1000)

    return updated_node_score, updated_repr
